# baseline (speedup 1.0000x reference)
import functools

import jax
import jax.numpy as jnp
from jax import lax
from jax.experimental import pallas as pl
from jax.experimental.pallas import tpu as pltpu
from jax.experimental.pallas import tpu_sc as plsc

N = 10000
D = 128
H = 128
A_OUT = 16
E = 320000

NC = 2
NS = 16
NW = NC * NS
K = 128
CPW = 80
EPW = K * CPW
E_PAD = EPW * NW
NPAD = 10240
DST_PAD = 10100
SLAB = NPAD // NS
NZC = SLAB // K

_mesh = plsc.VectorSubcoreMesh(
    core_axis_name="c", subcore_axis_name="s", num_cores=NC, num_subcores=NS
)


@functools.partial(
    pl.kernel,
    out_type=jax.ShapeDtypeStruct((NC * NPAD,), jnp.float32),
    mesh=_mesh,
    scratch_types=[
        pltpu.VMEM((CPW, K), jnp.int32),
        pltpu.VMEM((K,), jnp.float32),
        pltpu.VMEM((SLAB,), jnp.float32),
        pltpu.VMEM_SHARED((NPAD,), jnp.float32),
        pltpu.SemaphoreType.DMA,
        pltpu.SemaphoreType.DMA,
        pltpu.SemaphoreType.DMA,
        pltpu.SemaphoreType.DMA,
    ],
)
def _sc_degree(dst_hbm, out_hbm, dst_v, ones_v, stage_v, deg_sh,
               sem0, sem1, sem2, sem3):
    c = lax.axis_index("c")
    s = lax.axis_index("s")
    wid = s * NC + c

    def fill(i, _):
        ones_v[pl.ds(i * 16, 16)] = jnp.ones((16,), jnp.float32)
        return 0

    lax.fori_loop(0, K // 16, fill, 0)

    def zfill(i, _):
        stage_v[pl.ds(i * 16, 16)] = jnp.zeros((16,), jnp.float32)
        return 0

    lax.fori_loop(0, SLAB // 16, zfill, 0)
    pltpu.sync_copy(stage_v, deg_sh.at[pl.ds(s * SLAB, SLAB)])
    pltpu.sync_copy(dst_hbm.at[wid], dst_v)
    plsc.subcore_barrier()
    pl.delay(2000)
    plsc.subcore_barrier()

    def step(i, _):
        t = 4 * i
        d0 = pltpu.async_copy(ones_v, deg_sh.at[dst_v.at[t]], sem0, add=True)
        d1 = pltpu.async_copy(ones_v, deg_sh.at[dst_v.at[t + 1]], sem1, add=True)
        d2 = pltpu.async_copy(ones_v, deg_sh.at[dst_v.at[t + 2]], sem2, add=True)
        d3 = pltpu.async_copy(ones_v, deg_sh.at[dst_v.at[t + 3]], sem3, add=True)
        d0.wait()
        d1.wait()
        d2.wait()
        d3.wait()
        return 0

    lax.fori_loop(0, CPW // 4, step, 0)
    plsc.subcore_barrier()
    pl.delay(2000)
    plsc.subcore_barrier()
    pltpu.sync_copy(deg_sh.at[pl.ds(s * SLAB, SLAB)], stage_v)
    pltpu.sync_copy(stage_v, out_hbm.at[pl.ds(c * NPAD + s * SLAB, SLAB)])


@functools.partial(
    pl.kernel,
    out_type=jax.ShapeDtypeStruct((NC, NPAD, D), jnp.float32),
    mesh=_mesh,
    scratch_types=[
        pltpu.VMEM((K,), jnp.int32),
        pltpu.VMEM((K,), jnp.int32),
        pltpu.VMEM((K,), jnp.int32),
        pltpu.VMEM((K,), jnp.int32),
        pltpu.VMEM((K, D), jnp.float32),
        pltpu.VMEM((K, D), jnp.float32),
        pltpu.VMEM_SHARED((NPAD, D), jnp.float32),
        pltpu.SemaphoreType.DMA,
        pltpu.SemaphoreType.DMA,
    ],
)
def _sc_aggregate(g_hbm, src_hbm, dst_hbm, out_hbm, srcv0, srcv1,
                  dstv0, dstv1, rows0, rows1, agg_sh, g0, g1):
    c = lax.axis_index("c")
    s = lax.axis_index("s")
    wid = s * NC + c

    def zfill(i, _):
        rows0[i // 8, pl.ds((i % 8) * 16, 16)] = jnp.zeros((16,), jnp.float32)
        return 0

    lax.fori_loop(0, K * 8, zfill, 0)
    for b in range(NZC):
        pltpu.sync_copy(rows0, agg_sh.at[pl.ds(s * SLAB + b * K, K)])
    plsc.subcore_barrier()
    pl.delay(2000)
    plsc.subcore_barrier()

    def step(i, _):
        base0 = wid * EPW + 2 * i * K
        base1 = base0 + K
        pltpu.sync_copy(src_hbm.at[pl.ds(base0, K)], srcv0)
        pltpu.sync_copy(src_hbm.at[pl.ds(base1, K)], srcv1)
        d0 = pltpu.async_copy(g_hbm.at[srcv0], rows0, g0)
        d1 = pltpu.async_copy(g_hbm.at[srcv1], rows1, g1)
        pltpu.sync_copy(dst_hbm.at[pl.ds(base0, K)], dstv0)
        pltpu.sync_copy(dst_hbm.at[pl.ds(base1, K)], dstv1)
        d0.wait()
        pltpu.sync_copy(rows0, agg_sh.at[dstv0], add=True)
        d1.wait()
        pltpu.sync_copy(rows1, agg_sh.at[dstv1], add=True)
        return 0

    lax.fori_loop(0, CPW // 2, step, 0)
    plsc.subcore_barrier()
    pl.delay(2000)
    plsc.subcore_barrier()
    for b in range(NZC):
        pltpu.sync_copy(agg_sh.at[pl.ds(s * SLAB + b * K, K)], rows0)
        pltpu.sync_copy(rows0, out_hbm.at[c, pl.ds(s * SLAB + b * K, K)])


def _dinv_body(degs_ref, dinv_ref):
    dinv_ref[...] = lax.rsqrt(degs_ref[:NPAD] + degs_ref[NPAD:] + 1.0)


_dinv = pl.pallas_call(
    _dinv_body,
    out_shape=jax.ShapeDtypeStruct((NPAD,), jnp.float32),
)

MB = 1000


def _l1_body(x_ref, w_ref, dinv_ref, g_ref):
    h = jnp.dot(x_ref[...], w_ref[...], preferred_element_type=jnp.float32)
    g_ref[...] = h * dinv_ref[...]


_l1 = pl.pallas_call(
    _l1_body,
    grid=(N // MB,),
    in_specs=[
        pl.BlockSpec((MB, D), lambda i: (i, 0)),
        pl.BlockSpec((D, H), lambda i: (0, 0)),
        pl.BlockSpec((MB, 1), lambda i: (i, 0)),
    ],
    out_specs=pl.BlockSpec((MB, H), lambda i: (i, 0)),
    out_shape=jax.ShapeDtypeStruct((N, H), jnp.float32),
)


def _l2_body(agg_ref, g1_ref, dinv_ref, b1_ref, w2_ref, g2_ref):
    z = jnp.maximum(
        (agg_ref[0] + agg_ref[1] + g1_ref[...]) * dinv_ref[...] + b1_ref[...],
        0.0,
    )
    g2_ref[...] = (
        jnp.dot(z, w2_ref[...], preferred_element_type=jnp.float32)
        * dinv_ref[...]
    )


_l2 = pl.pallas_call(
    _l2_body,
    grid=(N // MB,),
    in_specs=[
        pl.BlockSpec((NC, MB, D), lambda i: (0, i, 0)),
        pl.BlockSpec((MB, H), lambda i: (i, 0)),
        pl.BlockSpec((MB, 1), lambda i: (i, 0)),
        pl.BlockSpec((1, H), lambda i: (0, 0)),
        pl.BlockSpec((H, H), lambda i: (0, 0)),
    ],
    out_specs=pl.BlockSpec((MB, H), lambda i: (i, 0)),
    out_shape=jax.ShapeDtypeStruct((N, H), jnp.float32),
)


def _head_body(agg_ref, g2_ref, dinv_ref, b2_ref, w3_ref, b3_ref, o_ref):
    z = jnp.maximum(
        (agg_ref[0] + agg_ref[1] + g2_ref[...]) * dinv_ref[...] + b2_ref[...],
        0.0,
    )
    o_ref[...] = (
        jnp.dot(z, w3_ref[...], preferred_element_type=jnp.float32)
        + b3_ref[...]
    )


_head = pl.pallas_call(
    _head_body,
    grid=(N // MB,),
    in_specs=[
        pl.BlockSpec((NC, MB, H), lambda i: (0, i, 0)),
        pl.BlockSpec((MB, H), lambda i: (i, 0)),
        pl.BlockSpec((MB, 1), lambda i: (i, 0)),
        pl.BlockSpec((1, H), lambda i: (0, 0)),
        pl.BlockSpec((H, A_OUT), lambda i: (0, 0)),
        pl.BlockSpec((1, A_OUT), lambda i: (0, 0)),
    ],
    out_specs=pl.BlockSpec((MB, A_OUT), lambda i: (i, 0)),
    out_shape=jax.ShapeDtypeStruct((N, A_OUT), jnp.float32),
)


def kernel(x, edge_index, W1, b1, W2, b2, W3, b3):
    src = edge_index[0].astype(jnp.int32)
    dst = edge_index[1].astype(jnp.int32)
    pad = E_PAD - E
    src_p = jnp.concatenate([src, jnp.zeros((pad,), jnp.int32)])
    dst_p = jnp.concatenate([dst, jnp.full((pad,), DST_PAD, jnp.int32)])
    dst3 = dst_p.reshape(NW, CPW, K)

    degs = _sc_degree(dst3)
    dinv = _dinv(degs)
    dinv_col = dinv[:N].reshape(N, 1)

    g1 = _l1(x, W1, dinv_col)
    agg1 = _sc_aggregate(g1, src_p, dst_p)
    g2 = _l2(agg1, g1, dinv_col, b1.reshape(1, H), W2)
    agg2 = _sc_aggregate(g2, src_p, dst_p)
    return _head(agg2, g2, dinv_col, b2.reshape(1, H), W3, b3.reshape(1, A_OUT))


# speedup vs baseline: 2.7043x; 2.7043x over previous
import functools

import jax
import jax.numpy as jnp
from jax import lax
from jax.experimental import pallas as pl
from jax.experimental.pallas import tpu as pltpu
from jax.experimental.pallas import tpu_sc as plsc

N = 10000
D = 128
H = 128
A_OUT = 16
E = 320000

NC = 2
NS = 16
NW = NC * NS
K = 128
CPW = 80
EPW = K * CPW
E_PAD = EPW * NW
NPAD = 10240
DST_PAD = 10100
SLAB = NPAD // NS
NZC = SLAB // K

_mesh = plsc.VectorSubcoreMesh(
    core_axis_name="c", subcore_axis_name="s", num_cores=NC, num_subcores=NS
)


@functools.partial(
    pl.kernel,
    out_type=jax.ShapeDtypeStruct((NC * NPAD,), jnp.float32),
    mesh=_mesh,
    scratch_types=[
        pltpu.VMEM((CPW, K), jnp.int32),
        pltpu.VMEM((K,), jnp.float32),
        pltpu.VMEM((SLAB,), jnp.float32),
        pltpu.VMEM_SHARED((NPAD,), jnp.float32),
        pltpu.SemaphoreType.DMA,
        pltpu.SemaphoreType.DMA,
        pltpu.SemaphoreType.DMA,
        pltpu.SemaphoreType.DMA,
    ],
)
def _sc_degree(dst_hbm, out_hbm, dst_v, ones_v, stage_v, deg_sh,
               sem0, sem1, sem2, sem3):
    c = lax.axis_index("c")
    s = lax.axis_index("s")
    wid = s * NC + c

    def fill(i, _):
        ones_v[pl.ds(i * 16, 16)] = jnp.ones((16,), jnp.float32)
        return 0

    lax.fori_loop(0, K // 16, fill, 0)

    def zfill(i, _):
        stage_v[pl.ds(i * 16, 16)] = jnp.zeros((16,), jnp.float32)
        return 0

    lax.fori_loop(0, SLAB // 16, zfill, 0)
    pltpu.sync_copy(stage_v, deg_sh.at[pl.ds(s * SLAB, SLAB)])
    pltpu.sync_copy(dst_hbm.at[wid], dst_v)
    plsc.subcore_barrier()
    pl.delay(2000)
    plsc.subcore_barrier()

    def step(i, _):
        t = 4 * i
        d0 = pltpu.async_copy(ones_v, deg_sh.at[dst_v.at[t]], sem0, add=True)
        d1 = pltpu.async_copy(ones_v, deg_sh.at[dst_v.at[t + 1]], sem1, add=True)
        d2 = pltpu.async_copy(ones_v, deg_sh.at[dst_v.at[t + 2]], sem2, add=True)
        d3 = pltpu.async_copy(ones_v, deg_sh.at[dst_v.at[t + 3]], sem3, add=True)
        d0.wait()
        d1.wait()
        d2.wait()
        d3.wait()
        return 0

    lax.fori_loop(0, CPW // 4, step, 0)
    plsc.subcore_barrier()
    pl.delay(2000)
    plsc.subcore_barrier()
    pltpu.sync_copy(deg_sh.at[pl.ds(s * SLAB, SLAB)], stage_v)
    pltpu.sync_copy(stage_v, out_hbm.at[pl.ds(c * NPAD + s * SLAB, SLAB)])


@functools.partial(
    pl.kernel,
    out_type=jax.ShapeDtypeStruct((NC, NPAD, D), jnp.float32),
    mesh=_mesh,
    scratch_types=[
        pltpu.VMEM((K,), jnp.int32),
        pltpu.VMEM((K,), jnp.int32),
        pltpu.VMEM((K,), jnp.int32),
        pltpu.VMEM((K,), jnp.int32),
        pltpu.VMEM((K, D), jnp.float32),
        pltpu.VMEM((K, D), jnp.float32),
        pltpu.VMEM_SHARED((NPAD, D), jnp.float32),
        pltpu.SemaphoreType.DMA,
        pltpu.SemaphoreType.DMA,
    ],
)
def _sc_aggregate(g_hbm, src_hbm, dst_hbm, out_hbm, srcv0, srcv1,
                  dstv0, dstv1, rows0, rows1, agg_sh, g0, g1):
    c = lax.axis_index("c")
    s = lax.axis_index("s")
    wid = s * NC + c

    def zfill(i, _):
        rows0[i // 8, pl.ds((i % 8) * 16, 16)] = jnp.zeros((16,), jnp.float32)
        return 0

    lax.fori_loop(0, K * 8, zfill, 0)
    for b in range(NZC):
        pltpu.sync_copy(rows0, agg_sh.at[pl.ds(s * SLAB + b * K, K)])
    plsc.subcore_barrier()
    pl.delay(2000)
    plsc.subcore_barrier()

    def step(i, _):
        base0 = wid * EPW + 2 * i * K
        base1 = base0 + K
        pltpu.sync_copy(src_hbm.at[pl.ds(base0, K)], srcv0)
        pltpu.sync_copy(src_hbm.at[pl.ds(base1, K)], srcv1)
        d0 = pltpu.async_copy(g_hbm.at[srcv0], rows0, g0)
        d1 = pltpu.async_copy(g_hbm.at[srcv1], rows1, g1)
        pltpu.sync_copy(dst_hbm.at[pl.ds(base0, K)], dstv0)
        pltpu.sync_copy(dst_hbm.at[pl.ds(base1, K)], dstv1)
        d0.wait()
        pltpu.sync_copy(rows0, agg_sh.at[dstv0], add=True)
        d1.wait()
        pltpu.sync_copy(rows1, agg_sh.at[dstv1], add=True)
        return 0

    lax.fori_loop(0, CPW // 2, step, 0)
    plsc.subcore_barrier()
    pl.delay(2000)
    plsc.subcore_barrier()
    for b in range(NZC):
        pltpu.sync_copy(agg_sh.at[pl.ds(s * SLAB + b * K, K)], rows0)
        pltpu.sync_copy(rows0, out_hbm.at[c, pl.ds(s * SLAB + b * K, K)])


def _dinv_body(degs_ref, dinv_ref):
    dinv_ref[...] = lax.rsqrt(degs_ref[:NPAD] + degs_ref[NPAD:] + 1.0)


_dinv = pl.pallas_call(
    _dinv_body,
    out_shape=jax.ShapeDtypeStruct((NPAD,), jnp.float32),
)

MB = 1000


def _l1_body(x_ref, w_ref, dinv_ref, g_ref):
    h = jnp.dot(x_ref[...], w_ref[...], preferred_element_type=jnp.float32)
    g_ref[...] = h * dinv_ref[...]


_l1 = pl.pallas_call(
    _l1_body,
    grid=(N // MB,),
    in_specs=[
        pl.BlockSpec((MB, D), lambda i: (i, 0)),
        pl.BlockSpec((D, H), lambda i: (0, 0)),
        pl.BlockSpec((MB, 1), lambda i: (i, 0)),
    ],
    out_specs=pl.BlockSpec((MB, H), lambda i: (i, 0)),
    out_shape=jax.ShapeDtypeStruct((N, H), jnp.float32),
)


def _l2_body(agg_ref, g1_ref, dinv_ref, b1_ref, w2_ref, g2_ref):
    z = jnp.maximum(
        (agg_ref[0] + agg_ref[1] + g1_ref[...]) * dinv_ref[...] + b1_ref[...],
        0.0,
    )
    g2_ref[...] = (
        jnp.dot(z, w2_ref[...], preferred_element_type=jnp.float32)
        * dinv_ref[...]
    )


_l2 = pl.pallas_call(
    _l2_body,
    grid=(N // MB,),
    in_specs=[
        pl.BlockSpec((NC, MB, D), lambda i: (0, i, 0)),
        pl.BlockSpec((MB, H), lambda i: (i, 0)),
        pl.BlockSpec((MB, 1), lambda i: (i, 0)),
        pl.BlockSpec((1, H), lambda i: (0, 0)),
        pl.BlockSpec((H, H), lambda i: (0, 0)),
    ],
    out_specs=pl.BlockSpec((MB, H), lambda i: (i, 0)),
    out_shape=jax.ShapeDtypeStruct((N, H), jnp.float32),
)


def _head_body(agg_ref, g2_ref, dinv_ref, b2_ref, w3_ref, b3_ref, o_ref):
    z = jnp.maximum(
        (agg_ref[0] + agg_ref[1] + g2_ref[...]) * dinv_ref[...] + b2_ref[...],
        0.0,
    )
    o_ref[...] = (
        jnp.dot(z, w3_ref[...], preferred_element_type=jnp.float32)
        + b3_ref[...]
    )


_head = pl.pallas_call(
    _head_body,
    grid=(N // MB,),
    in_specs=[
        pl.BlockSpec((NC, MB, H), lambda i: (0, i, 0)),
        pl.BlockSpec((MB, H), lambda i: (i, 0)),
        pl.BlockSpec((MB, 1), lambda i: (i, 0)),
        pl.BlockSpec((1, H), lambda i: (0, 0)),
        pl.BlockSpec((H, A_OUT), lambda i: (0, 0)),
        pl.BlockSpec((1, A_OUT), lambda i: (0, 0)),
    ],
    out_specs=pl.BlockSpec((MB, A_OUT), lambda i: (i, 0)),
    out_shape=jax.ShapeDtypeStruct((N, A_OUT), jnp.float32),
)


def kernel(x, edge_index, W1, b1, W2, b2, W3, b3):
    src = edge_index[0].astype(jnp.int32)
    dst = edge_index[1].astype(jnp.int32)
    pad = E_PAD - E
    pad_idx = jnp.arange(pad, dtype=jnp.int32)
    src_p = jnp.concatenate([src, pad_idx % 128])
    dst_p = jnp.concatenate([dst, N + (pad_idx % (NPAD - N))])
    dst3 = dst_p.reshape(NW, CPW, K)

    degs = _sc_degree(dst3)
    dinv = _dinv(degs)
    dinv_col = dinv[:N].reshape(N, 1)

    g1 = _l1(x, W1, dinv_col)
    agg1 = _sc_aggregate(g1, src_p, dst_p)
    g2 = _l2(agg1, g1, dinv_col, b1.reshape(1, H), W2)
    agg2 = _sc_aggregate(g2, src_p, dst_p)
    return _head(agg2, g2, dinv_col, b2.reshape(1, H), W3, b3.reshape(1, A_OUT))
